# baseline (speedup 1.0000x reference)
import functools

import jax
import jax.numpy as jnp
from jax import lax
from jax.experimental import pallas as pl
from jax.experimental.pallas import tpu as pltpu
from jax.experimental.pallas import tpu_sc as plsc

T = 2048
D = 1024
DFF = 1408
E = 64
NG = 8
GSZ = E // NG
TKG = 3
TK = 8
DSH = 2 * DFF
SCALE = 1.0

A = T * TK
R = 256
NT = A // R + E
NPAD = NT * R

_F32 = jnp.float32
_BF16 = jnp.bfloat16
_I32 = jnp.int32


def _router_body(x_ref, wg_ref, tw_ref, dest_ref, ett_ref, nr_ref):
    x = x_ref[...]
    wg = wg_ref[...]
    logits = lax.dot_general(
        x, wg, (((1,), (1,)), ((), ())),
        preferred_element_type=_F32, precision=lax.Precision.HIGHEST)
    m = jnp.max(logits, axis=1, keepdims=True)
    ex = jnp.exp(logits - m)
    s = ex / jnp.sum(ex, axis=1, keepdims=True)

    gcols = [jnp.max(s[:, g * GSZ:(g + 1) * GSZ], axis=1, keepdims=True)
             for g in range(NG)]
    gs = jnp.concatenate(gcols, axis=1)

    iota_g = lax.broadcasted_iota(_I32, (T, NG), 1)
    gmask = jnp.zeros((T, NG), dtype=jnp.bool_)
    gwork = gs
    for _ in range(TKG):
        mv = jnp.max(gwork, axis=1, keepdims=True)
        first = jnp.min(jnp.where(gwork >= mv, iota_g, NG), axis=1,
                        keepdims=True)
        sel = iota_g == first
        gmask = jnp.logical_or(gmask, sel)
        gwork = jnp.where(sel, -1.0, gwork)

    iota_e = lax.broadcasted_iota(_I32, (T, E), 1)
    grp_of_lane = iota_e // GSZ
    smask = jnp.zeros((T, E), dtype=jnp.bool_)
    for g in range(NG):
        smask = jnp.logical_or(
            smask, jnp.logical_and(gmask[:, g:g + 1], grp_of_lane == g))
    masked = jnp.where(smask, s, 0.0)

    work = masked
    sels = []
    for k in range(TK):
        mv = jnp.max(work, axis=1, keepdims=True)
        first = jnp.min(jnp.where(work >= mv, iota_e, E), axis=1,
                        keepdims=True)
        sel = iota_e == first
        sels.append(sel)
        tw_ref[:, k:k + 1] = mv
        work = jnp.where(sel, -1.0, work)

    cnt = jnp.zeros((T, E), dtype=_I32)
    for sel in sels:
        cnt = cnt + sel.astype(_I32)
    cinc = cnt
    sh = 1
    while sh < T:
        cinc = cinc + jnp.concatenate(
            [jnp.zeros((sh, E), _I32), cinc[:T - sh]], axis=0)
        sh *= 2
    cexc = cinc - cnt
    ct = cinc[T - 1:T, :]
    ntile = (ct + (R - 1)) // R
    tinc = ntile
    sh = 1
    while sh < E:
        tinc = tinc + jnp.concatenate(
            [jnp.zeros((1, sh), _I32), tinc[:, :E - sh]], axis=1)
        sh *= 2
    tstart = tinc - ntile
    tend = tinc
    offpad = tstart * R

    base = offpad + cexc
    for k, sel in enumerate(sels):
        dest_ref[:, k:k + 1] = jnp.sum(
            jnp.where(sel, base, 0), axis=1, keepdims=True)

    tid = lax.broadcasted_iota(_I32, (NT, 1), 0)
    tid_b = lax.broadcasted_iota(_I32, (NT, E), 0)
    ett = jnp.sum((tid_b >= tend).astype(_I32), axis=1, keepdims=True)
    ett = jnp.minimum(ett, E - 1)
    onehot = lax.broadcasted_iota(_I32, (NT, E), 1) == ett
    ct_t = jnp.sum(jnp.where(onehot, ct, 0), axis=1, keepdims=True)
    ts_t = jnp.sum(jnp.where(onehot, tstart, 0), axis=1, keepdims=True)
    nr = jnp.clip(ct_t - (tid - ts_t) * R, 0, R)
    ett_ref[...] = ett
    nr_ref[...] = nr


def _router(x, Wg):
    return pl.pallas_call(
        _router_body,
        out_shape=[
            jax.ShapeDtypeStruct((T, TK), _F32),
            jax.ShapeDtypeStruct((T, TK), _I32),
            jax.ShapeDtypeStruct((NT, 1), _I32),
            jax.ShapeDtypeStruct((NT, 1), _I32),
        ],
    )(x, Wg)


_NC = 2
_NS = 16
_NW = _NC * _NS
_PW = A // _NW
_CH = 32
_NCH = _PW // _CH


_TPW = T // _NW
_HTP = _TPW // 2


def _dispatch_sc(x, dest_t):
    mesh = plsc.VectorSubcoreMesh(core_axis_name="c", subcore_axis_name="s")

    @functools.partial(
        pl.kernel,
        out_type=jax.ShapeDtypeStruct((NPAD, D), _F32),
        mesh=mesh,
        scratch_types=[
            pltpu.VMEM((TK, _TPW), _I32),
            pltpu.VMEM((_HTP, D), _F32),
            pltpu.VMEM((_HTP, D), _F32),
            pltpu.SemaphoreType.DMA,
            pltpu.SemaphoreType.DMA,
        ],
    )
    def k2(x_hbm, dest_hbm, xs_hbm, dest_v, xb0, xb1, sem0, sem1):
        wid = lax.axis_index("s") * _NC + lax.axis_index("c")
        tbase = wid * _TPW
        for k in range(TK):
            pltpu.sync_copy(dest_hbm.at[pl.ds(k * T + tbase, _TPW)],
                            dest_v.at[k])
        ld0 = pltpu.make_async_copy(x_hbm.at[pl.ds(tbase, _HTP)], xb0, sem0)
        ld1 = pltpu.make_async_copy(
            x_hbm.at[pl.ds(tbase + _HTP, _HTP)], xb1, sem1)
        ld0.start()
        ld1.start()
        ld0.wait()
        sc0 = [pltpu.make_async_copy(
                   xb0, xs_hbm.at[dest_v.at[k, pl.ds(0, _HTP)]], sem0)
               for k in range(TK)]
        for s in sc0:
            s.start()
        ld1.wait()
        sc1 = [pltpu.make_async_copy(
                   xb1, xs_hbm.at[dest_v.at[k, pl.ds(_HTP, _HTP)]], sem1)
               for k in range(TK)]
        for s in sc1:
            s.start()
        for s in sc0:
            s.wait()
        for s in sc1:
            s.wait()

    return k2(x, dest_t)


def _combine_gather_sc(ys, dest):
    mesh = plsc.VectorSubcoreMesh(core_axis_name="c", subcore_axis_name="s")

    @functools.partial(
        pl.kernel,
        out_type=jax.ShapeDtypeStruct((A, D), _F32),
        mesh=mesh,
        scratch_types=[
            pltpu.VMEM((2, _CH), _I32),
            pltpu.VMEM((_CH, D), _F32),
            pltpu.VMEM((_CH, D), _F32),
            pltpu.SemaphoreType.DMA,
            pltpu.SemaphoreType.DMA,
        ],
    )
    def k4(ys_hbm, dest_hbm, yg_hbm, dest_v, rows0, rows1, sem0, sem1):
        wid = lax.axis_index("s") * _NC + lax.axis_index("c")
        base = wid * _PW
        rows = (rows0, rows1)
        sems = (sem0, sem1)

        def issue(slot, ci):
            off = base + ci * _CH
            pltpu.sync_copy(dest_hbm.at[pl.ds(off, _CH)], dest_v.at[slot])
            pltpu.make_async_copy(
                ys_hbm.at[dest_v.at[slot]], rows[slot], sems[slot]).start()

        issue(0, 0)
        issue(1, 1)

        @pl.loop(0, _NCH, step=2)
        def _(ci):
            for b in range(2):
                pltpu.make_async_copy(
                    ys_hbm.at[dest_v.at[b]], rows[b], sems[b]).wait()
                off = base + (ci + b) * _CH
                pltpu.sync_copy(rows[b], yg_hbm.at[pl.ds(off, _CH)])

                @pl.when(ci + b + 2 < _NCH)
                def _():
                    issue(b, ci + b + 2)

    return k4(ys, dest)


def _gemm_body(ett_ref, nr_ref, xs_ref, wg_ref, wu_ref, wd_ref, ys_ref,
               wgb_ref, wub_ref, wdb_ref):
    t = pl.program_id(0)

    @pl.when(nr_ref[t] > 0)
    def _():
        e = ett_ref[t]
        prev_e = ett_ref[jnp.maximum(t - 1, 0)]
        changed = jnp.logical_or(t == 0, e != prev_e)

        @pl.when(changed)
        def _():
            wgb_ref[...] = wg_ref[0].astype(_BF16)
            wub_ref[...] = wu_ref[0].astype(_BF16)
            wdb_ref[...] = wd_ref[0].astype(_BF16)

        xb = xs_ref[...].astype(_BF16)
        g = lax.dot_general(xb, wgb_ref[...], (((1,), (1,)), ((), ())),
                            preferred_element_type=_F32)
        u = lax.dot_general(xb, wub_ref[...], (((1,), (1,)), ((), ())),
                            preferred_element_type=_F32)
        h = (g * jax.nn.sigmoid(g) * u).astype(_BF16)
        ys_ref[...] = lax.dot_general(h, wdb_ref[...], (((1,), (1,)), ((), ())),
                                      preferred_element_type=_F32)


def _grouped_gemm(ett, nr, xs, We_gate, We_up, We_down):
    grid_spec = pltpu.PrefetchScalarGridSpec(
        num_scalar_prefetch=2,
        grid=(NT,),
        in_specs=[
            pl.BlockSpec((R, D),
                         lambda t, ett, nr: (jnp.where(nr[t] > 0, t, NT - 1), 0)),
            pl.BlockSpec((1, DFF, D), lambda t, ett, nr: (ett[t], 0, 0)),
            pl.BlockSpec((1, DFF, D), lambda t, ett, nr: (ett[t], 0, 0)),
            pl.BlockSpec((1, D, DFF), lambda t, ett, nr: (ett[t], 0, 0)),
        ],
        out_specs=pl.BlockSpec(
            (R, D), lambda t, ett, nr: (jnp.where(nr[t] > 0, t, NT - 1), 0)),
        scratch_shapes=[
            pltpu.VMEM((DFF, D), _BF16),
            pltpu.VMEM((DFF, D), _BF16),
            pltpu.VMEM((D, DFF), _BF16),
        ],
    )
    return pl.pallas_call(
        _gemm_body,
        grid_spec=grid_spec,
        out_shape=jax.ShapeDtypeStruct((NPAD, D), _F32),
    )(ett, nr, xs, We_gate, We_up, We_down)


_CSH = 256
_NCSH = DSH // _CSH


def _shared_body(x_ref, wsg_ref, wsu_ref, wsd_ref, out_ref, xb_ref):
    c = pl.program_id(0)

    @pl.when(c == 0)
    def _():
        xb_ref[...] = x_ref[...].astype(_BF16)

    xb = xb_ref[...]
    wsg = wsg_ref[...].astype(_BF16)
    wsu = wsu_ref[...].astype(_BF16)
    wsd = wsd_ref[...].astype(_BF16)
    g = lax.dot_general(xb, wsg, (((1,), (1,)), ((), ())),
                        preferred_element_type=_F32)
    u = lax.dot_general(xb, wsu, (((1,), (1,)), ((), ())),
                        preferred_element_type=_F32)
    h = (g * jax.nn.sigmoid(g) * u).astype(_BF16)
    y = lax.dot_general(h, wsd, (((1,), (1,)), ((), ())),
                        preferred_element_type=_F32)

    @pl.when(c == 0)
    def _():
        out_ref[...] = y

    @pl.when(c != 0)
    def _():
        out_ref[...] = out_ref[...] + y


def _shared_mlp(x, Ws_gate, Ws_up, Ws_down):
    return pl.pallas_call(
        _shared_body,
        grid=(_NCSH,),
        in_specs=[
            pl.BlockSpec((T, D), lambda c: (0, 0)),
            pl.BlockSpec((_CSH, D), lambda c: (c, 0)),
            pl.BlockSpec((_CSH, D), lambda c: (c, 0)),
            pl.BlockSpec((D, _CSH), lambda c: (0, c)),
        ],
        out_specs=pl.BlockSpec((T, D), lambda c: (0, 0)),
        out_shape=jax.ShapeDtypeStruct((T, D), _F32),
        scratch_shapes=[pltpu.VMEM((T, D), _BF16)],
    )(x, Ws_gate, Ws_up, Ws_down)


_TT = 256
_NTT = T // _TT


def _combine_body(yg_ref, tw_ref, sh_ref, out_ref):
    tw = tw_ref[...]
    acc = sh_ref[...]
    for k in range(TK):
        acc = acc + (SCALE * tw[:, k:k + 1]) * yg_ref[:, k * D:(k + 1) * D]
    out_ref[...] = acc


def _combine(yg, tw, shared):
    return pl.pallas_call(
        _combine_body,
        grid=(_NTT,),
        in_specs=[
            pl.BlockSpec((_TT, TK * D), lambda t: (t, 0)),
            pl.BlockSpec((_TT, TK), lambda t: (t, 0)),
            pl.BlockSpec((_TT, D), lambda t: (t, 0)),
        ],
        out_specs=pl.BlockSpec((_TT, D), lambda t: (t, 0)),
        out_shape=jax.ShapeDtypeStruct((T, D), _F32),
    )(yg.reshape(T, TK * D), tw, shared)


def kernel(x, Wg, We_gate, We_up, We_down, Ws_gate, Ws_up, Ws_down):
    tw, dest2, ett2, nr2 = _router(x, Wg)
    dest = dest2.reshape(A)
    ett = ett2.reshape(NT)
    nr = nr2.reshape(NT)
    xs = _dispatch_sc(x, dest2.T.reshape(TK * T))
    shared = _shared_mlp(x, Ws_gate, Ws_up, Ws_down)
    ys = _grouped_gemm(ett, nr, xs, We_gate, We_up, We_down)
    yg = _combine_gather_sc(ys, dest)
    return _combine(yg, tw, shared)


# speedup vs baseline: 1.0637x; 1.0637x over previous
import functools

import jax
import jax.numpy as jnp
from jax import lax
from jax.experimental import pallas as pl
from jax.experimental.pallas import tpu as pltpu
from jax.experimental.pallas import tpu_sc as plsc

T = 2048
D = 1024
DFF = 1408
E = 64
NG = 8
GSZ = E // NG
TKG = 3
TK = 8
DSH = 2 * DFF
SCALE = 1.0

A = T * TK
R = 256
NT = A // R + E
NPAD = NT * R

_F32 = jnp.float32
_BF16 = jnp.bfloat16
_I32 = jnp.int32


def _router_body(x_ref, wg_ref, tw_ref, dest_ref, ett_ref, nr_ref):
    x = x_ref[...]
    wg = wg_ref[...]
    logits = lax.dot_general(
        x, wg, (((1,), (1,)), ((), ())),
        preferred_element_type=_F32, precision=lax.Precision.HIGHEST)
    m = jnp.max(logits, axis=1, keepdims=True)
    ex = jnp.exp(logits - m)
    s = ex / jnp.sum(ex, axis=1, keepdims=True)

    gcols = [jnp.max(s[:, g * GSZ:(g + 1) * GSZ], axis=1, keepdims=True)
             for g in range(NG)]
    gs = jnp.concatenate(gcols, axis=1)

    iota_g = lax.broadcasted_iota(_I32, (T, NG), 1)
    gmask = jnp.zeros((T, NG), dtype=jnp.bool_)
    gwork = gs
    for _ in range(TKG):
        mv = jnp.max(gwork, axis=1, keepdims=True)
        first = jnp.min(jnp.where(gwork >= mv, iota_g, NG), axis=1,
                        keepdims=True)
        sel = iota_g == first
        gmask = jnp.logical_or(gmask, sel)
        gwork = jnp.where(sel, -1.0, gwork)

    iota_e = lax.broadcasted_iota(_I32, (T, E), 1)
    grp_of_lane = iota_e // GSZ
    smask = jnp.zeros((T, E), dtype=jnp.bool_)
    for g in range(NG):
        smask = jnp.logical_or(
            smask, jnp.logical_and(gmask[:, g:g + 1], grp_of_lane == g))
    masked = jnp.where(smask, s, 0.0)

    work = masked
    sels = []
    for k in range(TK):
        mv = jnp.max(work, axis=1, keepdims=True)
        first = jnp.min(jnp.where(work >= mv, iota_e, E), axis=1,
                        keepdims=True)
        sel = iota_e == first
        sels.append(sel)
        tw_ref[:, k:k + 1] = mv
        work = jnp.where(sel, -1.0, work)

    cnt = jnp.zeros((T, E), dtype=_I32)
    for sel in sels:
        cnt = cnt + sel.astype(_I32)
    cinc = cnt
    sh = 1
    while sh < T:
        cinc = cinc + jnp.concatenate(
            [jnp.zeros((sh, E), _I32), cinc[:T - sh]], axis=0)
        sh *= 2
    cexc = cinc - cnt
    ct = cinc[T - 1:T, :]
    ntile = (ct + (R - 1)) // R
    tinc = ntile
    sh = 1
    while sh < E:
        tinc = tinc + jnp.concatenate(
            [jnp.zeros((1, sh), _I32), tinc[:, :E - sh]], axis=1)
        sh *= 2
    tstart = tinc - ntile
    tend = tinc
    offpad = tstart * R

    base = offpad + cexc
    for k, sel in enumerate(sels):
        dest_ref[:, k:k + 1] = jnp.sum(
            jnp.where(sel, base, 0), axis=1, keepdims=True)

    tid = lax.broadcasted_iota(_I32, (NT, 1), 0)
    tid_b = lax.broadcasted_iota(_I32, (NT, E), 0)
    ett = jnp.sum((tid_b >= tend).astype(_I32), axis=1, keepdims=True)
    ett = jnp.minimum(ett, E - 1)
    onehot = lax.broadcasted_iota(_I32, (NT, E), 1) == ett
    ct_t = jnp.sum(jnp.where(onehot, ct, 0), axis=1, keepdims=True)
    ts_t = jnp.sum(jnp.where(onehot, tstart, 0), axis=1, keepdims=True)
    nr = jnp.clip(ct_t - (tid - ts_t) * R, 0, R)
    ett_ref[...] = ett
    nr_ref[...] = nr


def _router(x, Wg):
    return pl.pallas_call(
        _router_body,
        out_shape=[
            jax.ShapeDtypeStruct((T, TK), _F32),
            jax.ShapeDtypeStruct((T, TK), _I32),
            jax.ShapeDtypeStruct((NT, 1), _I32),
            jax.ShapeDtypeStruct((NT, 1), _I32),
        ],
    )(x, Wg)


_NC = 2
_NS = 16
_NW = _NC * _NS
_PW = A // _NW
_CH = 32
_NCH = _PW // _CH


_TPW = T // _NW
_HTP = _TPW // 2


def _dispatch_sc(x, dest_t):
    mesh = plsc.VectorSubcoreMesh(core_axis_name="c", subcore_axis_name="s")

    @functools.partial(
        pl.kernel,
        out_type=jax.ShapeDtypeStruct((NPAD, D), _F32),
        mesh=mesh,
        scratch_types=[
            pltpu.VMEM((TK, _TPW), _I32),
            pltpu.VMEM((_HTP, D), _F32),
            pltpu.VMEM((_HTP, D), _F32),
            pltpu.SemaphoreType.DMA,
            pltpu.SemaphoreType.DMA,
        ],
    )
    def k2(x_hbm, dest_hbm, xs_hbm, dest_v, xb0, xb1, sem0, sem1):
        wid = lax.axis_index("s") * _NC + lax.axis_index("c")
        tbase = wid * _TPW
        for k in range(TK):
            pltpu.sync_copy(dest_hbm.at[pl.ds(k * T + tbase, _TPW)],
                            dest_v.at[k])
        ld0 = pltpu.make_async_copy(x_hbm.at[pl.ds(tbase, _HTP)], xb0, sem0)
        ld1 = pltpu.make_async_copy(
            x_hbm.at[pl.ds(tbase + _HTP, _HTP)], xb1, sem1)
        ld0.start()
        ld1.start()
        ld0.wait()
        sc0 = [pltpu.make_async_copy(
                   xb0, xs_hbm.at[dest_v.at[k, pl.ds(0, _HTP)]], sem0)
               for k in range(TK)]
        for s in sc0:
            s.start()
        ld1.wait()
        sc1 = [pltpu.make_async_copy(
                   xb1, xs_hbm.at[dest_v.at[k, pl.ds(_HTP, _HTP)]], sem1)
               for k in range(TK)]
        for s in sc1:
            s.start()
        for s in sc0:
            s.wait()
        for s in sc1:
            s.wait()

    return k2(x, dest_t)


def _combine_gather_sc(ys, dest):
    mesh = plsc.VectorSubcoreMesh(core_axis_name="c", subcore_axis_name="s")

    @functools.partial(
        pl.kernel,
        out_type=jax.ShapeDtypeStruct((A, D), _F32),
        mesh=mesh,
        scratch_types=[
            pltpu.VMEM((2, _CH), _I32),
            pltpu.VMEM((_CH, D), _F32),
            pltpu.VMEM((_CH, D), _F32),
            pltpu.SemaphoreType.DMA,
            pltpu.SemaphoreType.DMA,
        ],
    )
    def k4(ys_hbm, dest_hbm, yg_hbm, dest_v, rows0, rows1, sem0, sem1):
        wid = lax.axis_index("s") * _NC + lax.axis_index("c")
        base = wid * _PW
        rows = (rows0, rows1)
        sems = (sem0, sem1)

        def issue(slot, ci):
            off = base + ci * _CH
            pltpu.sync_copy(dest_hbm.at[pl.ds(off, _CH)], dest_v.at[slot])
            pltpu.make_async_copy(
                ys_hbm.at[dest_v.at[slot]], rows[slot], sems[slot]).start()

        issue(0, 0)
        issue(1, 1)

        @pl.loop(0, _NCH, step=2)
        def _(ci):
            for b in range(2):
                pltpu.make_async_copy(
                    ys_hbm.at[dest_v.at[b]], rows[b], sems[b]).wait()
                off = base + (ci + b) * _CH
                pltpu.sync_copy(rows[b], yg_hbm.at[pl.ds(off, _CH)])

                @pl.when(ci + b + 2 < _NCH)
                def _():
                    issue(b, ci + b + 2)

    return k4(ys, dest)


def _gemm_body(ett_ref, nr_ref, xs_ref, wg_ref, wu_ref, wd_ref, ys_ref,
               wgb_ref, wub_ref, wdb_ref):
    t = pl.program_id(0)

    @pl.when(nr_ref[t] > 0)
    def _():
        e = ett_ref[t]
        prev_e = ett_ref[jnp.maximum(t - 1, 0)]
        changed = jnp.logical_or(t == 0, e != prev_e)

        @pl.when(changed)
        def _():
            wgb_ref[...] = wg_ref[0].astype(_BF16)
            wub_ref[...] = wu_ref[0].astype(_BF16)
            wdb_ref[...] = wd_ref[0].astype(_BF16)

        xb = xs_ref[...].astype(_BF16)
        g = lax.dot_general(xb, wgb_ref[...], (((1,), (1,)), ((), ())),
                            preferred_element_type=_F32)
        u = lax.dot_general(xb, wub_ref[...], (((1,), (1,)), ((), ())),
                            preferred_element_type=_F32)
        h = (g * jax.nn.sigmoid(g) * u).astype(_BF16)
        ys_ref[...] = lax.dot_general(h, wdb_ref[...], (((1,), (1,)), ((), ())),
                                      preferred_element_type=_F32)


def _grouped_gemm(ett, nr, xs, We_gate, We_up, We_down):
    grid_spec = pltpu.PrefetchScalarGridSpec(
        num_scalar_prefetch=2,
        grid=(NT,),
        in_specs=[
            pl.BlockSpec((R, D),
                         lambda t, ett, nr: (jnp.where(nr[t] > 0, t, NT - 1), 0)),
            pl.BlockSpec((1, DFF, D), lambda t, ett, nr: (ett[t], 0, 0)),
            pl.BlockSpec((1, DFF, D), lambda t, ett, nr: (ett[t], 0, 0)),
            pl.BlockSpec((1, D, DFF), lambda t, ett, nr: (ett[t], 0, 0)),
        ],
        out_specs=pl.BlockSpec(
            (R, D), lambda t, ett, nr: (jnp.where(nr[t] > 0, t, NT - 1), 0)),
        scratch_shapes=[
            pltpu.VMEM((DFF, D), _BF16),
            pltpu.VMEM((DFF, D), _BF16),
            pltpu.VMEM((D, DFF), _BF16),
        ],
    )
    return pl.pallas_call(
        _gemm_body,
        grid_spec=grid_spec,
        out_shape=jax.ShapeDtypeStruct((NPAD, D), _F32),
    )(ett, nr, xs, We_gate, We_up, We_down)


_CSH = 256
_NCSH = DSH // _CSH


def _shared_body(x_ref, wsg_ref, wsu_ref, wsd_ref, out_ref, xb_ref):
    c = pl.program_id(0)

    @pl.when(c == 0)
    def _():
        xb_ref[...] = x_ref[...].astype(_BF16)

    xb = xb_ref[...]
    wsg = wsg_ref[...].astype(_BF16)
    wsu = wsu_ref[...].astype(_BF16)
    wsd = wsd_ref[...].astype(_BF16)
    g = lax.dot_general(xb, wsg, (((1,), (1,)), ((), ())),
                        preferred_element_type=_F32)
    u = lax.dot_general(xb, wsu, (((1,), (1,)), ((), ())),
                        preferred_element_type=_F32)
    h = (g * jax.nn.sigmoid(g) * u).astype(_BF16)
    y = lax.dot_general(h, wsd, (((1,), (1,)), ((), ())),
                        preferred_element_type=_F32)

    @pl.when(c == 0)
    def _():
        out_ref[...] = y

    @pl.when(c != 0)
    def _():
        out_ref[...] = out_ref[...] + y


def _shared_mlp(x, Ws_gate, Ws_up, Ws_down):
    return pl.pallas_call(
        _shared_body,
        grid=(_NCSH,),
        in_specs=[
            pl.BlockSpec((T, D), lambda c: (0, 0)),
            pl.BlockSpec((_CSH, D), lambda c: (c, 0)),
            pl.BlockSpec((_CSH, D), lambda c: (c, 0)),
            pl.BlockSpec((D, _CSH), lambda c: (0, c)),
        ],
        out_specs=pl.BlockSpec((T, D), lambda c: (0, 0)),
        out_shape=jax.ShapeDtypeStruct((T, D), _F32),
        scratch_shapes=[pltpu.VMEM((T, D), _BF16)],
    )(x, Ws_gate, Ws_up, Ws_down)


_TT = 256
_NTT = T // _TT


def _combine_body(yg_ref, tw_ref, sh_ref, out_ref):
    k = pl.program_id(1)
    lane = lax.broadcasted_iota(_I32, (_TT, TK), 1)
    wk = jnp.sum(jnp.where(lane == k, tw_ref[...], 0.0), axis=1,
                 keepdims=True)
    y = (SCALE * wk) * yg_ref[...]

    @pl.when(k == 0)
    def _():
        out_ref[...] = sh_ref[...] + y

    @pl.when(k != 0)
    def _():
        out_ref[...] = out_ref[...] + y


def _combine(yg, tw, shared):
    return pl.pallas_call(
        _combine_body,
        grid=(_NTT, TK),
        in_specs=[
            pl.BlockSpec((_TT, D), lambda t, k: (k * _NTT + t, 0)),
            pl.BlockSpec((_TT, TK), lambda t, k: (t, 0)),
            pl.BlockSpec((_TT, D), lambda t, k: (t, 0)),
        ],
        out_specs=pl.BlockSpec((_TT, D), lambda t, k: (t, 0)),
        out_shape=jax.ShapeDtypeStruct((T, D), _F32),
    )(yg, tw, shared)


def kernel(x, Wg, We_gate, We_up, We_down, Ws_gate, Ws_up, Ws_down):
    tw, dest2, ett2, nr2 = _router(x, Wg)
    dest_t = dest2.T.reshape(TK * T)
    ett = ett2.reshape(NT)
    nr = nr2.reshape(NT)
    xs = _dispatch_sc(x, dest_t)
    shared = _shared_mlp(x, Ws_gate, Ws_up, Ws_down)
    ys = _grouped_gemm(ett, nr, xs, We_gate, We_up, We_down)
    yg = _combine_gather_sc(ys, dest_t)
    return _combine(yg, tw, shared)
